# Initial kernel scaffold; baseline (speedup 1.0000x reference)
import jax
import jax.numpy as jnp
from jax.experimental import pallas as pl


def kernel(q, k, bucket_size, topk):
    raise NotImplementedError("write your pallas kernel here")


# speedup vs baseline: 2.8868x; 2.8868x over previous
import functools

import jax
import jax.numpy as jnp
from jax import lax
from jax.experimental import pallas as pl
from jax.experimental.pallas import tpu as pltpu
from jax.experimental.pallas import tpu_sc as plsc

_DIM = 128
_NC = 2
_NS = 16
_CHUNK = 64
_NBUF = 4


def _segsum_body(q_hbm, k_hbm, seg_hbm, z_hbm, qs_hbm, ks_hbm,
                 idx_v, b0, b1, b2, b3, accq_sh, acck_sh, s0, s1, s2, s3):
    c = lax.axis_index("c")
    s = lax.axis_index("s")
    wid = s * _NC + c
    bufs = (b0, b1, b2, b3)
    sems = (s0, s1, s2, s3)

    pltpu.sync_copy(seg_hbm.at[s], idx_v)
    pltpu.sync_copy(z_hbm, accq_sh.at[pl.ds(s * 128, 128)])
    pltpu.sync_copy(z_hbm, acck_sh.at[pl.ds(s * 128, 128)])

    n_chunks = q_hbm.shape[1] // _CHUNK
    full_groups = n_chunks // _NBUF
    tail = n_chunks % _NBUF

    def stream_tensor(x_hbm, acc_sh):
        for b in range(_NBUF):
            pltpu.async_copy(x_hbm.at[wid, pl.ds(b * _CHUNK, _CHUNK)],
                             bufs[b], sems[b])

        def outer(g, carry):
            for b in range(_NBUF):
                cc = g * _NBUF + b
                pltpu.make_async_copy(
                    x_hbm.at[wid, pl.ds(cc * _CHUNK, _CHUNK)],
                    bufs[b], sems[b]).wait()
                pltpu.sync_copy(bufs[b], acc_sh.at[idx_v.at[cc]], add=True)
                nxt = cc + _NBUF

                @pl.when(nxt < n_chunks)
                def _():
                    pltpu.async_copy(x_hbm.at[wid, pl.ds(nxt * _CHUNK, _CHUNK)],
                                     bufs[b], sems[b])
            return carry

        lax.fori_loop(0, full_groups, outer, 0)
        for b in range(tail):
            cc = full_groups * _NBUF + b
            pltpu.make_async_copy(
                x_hbm.at[wid, pl.ds(cc * _CHUNK, _CHUNK)],
                bufs[b], sems[b]).wait()
            pltpu.sync_copy(bufs[b], acc_sh.at[idx_v.at[cc]], add=True)

    stream_tensor(q_hbm, accq_sh)
    stream_tensor(k_hbm, acck_sh)

    pltpu.sync_copy(accq_sh.at[pl.ds(s * 128, 128)], qs_hbm.at[wid])
    pltpu.sync_copy(acck_sh.at[pl.ds(s * 128, 128)], ks_hbm.at[wid])


def _segsum_sc(q, k, seg_off, zeros):
    b_h, t, d = q.shape
    L = 128
    mesh = plsc.VectorSubcoreMesh(core_axis_name="c", subcore_axis_name="s")
    f = pl.kernel(
        _segsum_body,
        out_type=[jax.ShapeDtypeStruct((b_h, L, d), jnp.float32),
                  jax.ShapeDtypeStruct((b_h, L, d), jnp.float32)],
        mesh=mesh,
        scratch_types=[
            pltpu.VMEM((t // _CHUNK, _CHUNK), jnp.int32),
            pltpu.VMEM((_CHUNK, d), jnp.float32),
            pltpu.VMEM((_CHUNK, d), jnp.float32),
            pltpu.VMEM((_CHUNK, d), jnp.float32),
            pltpu.VMEM((_CHUNK, d), jnp.float32),
            pltpu.VMEM_SHARED((_NS * L, d), jnp.float32),
            pltpu.VMEM_SHARED((_NS * L, d), jnp.float32),
            pltpu.SemaphoreType.DMA,
            pltpu.SemaphoreType.DMA,
            pltpu.SemaphoreType.DMA,
            pltpu.SemaphoreType.DMA,
        ],
    )
    return f(q, k, seg_off, zeros)


def _finish_kernel(topk_ref, qs_ref, ks_ref, out_ref, *, L):
    qs = qs_ref[0].astype(jnp.bfloat16)
    ks = ks_ref[0].astype(jnp.bfloat16)
    R = jax.lax.dot_general(qs, ks, (((1,), (1,)), ((), ())),
                            preferred_element_type=jnp.float32)
    R = R * jnp.float32(_DIM ** -0.5)
    R = R * topk_ref[...]
    m = jnp.max(R, axis=-1, keepdims=True)
    e = jnp.exp(R - m)
    ssum = jnp.sum(e, axis=-1, keepdims=True)
    sm = e / ssum
    msm = jnp.max(sm, axis=-1, keepdims=True)
    lidx = jax.lax.broadcasted_iota(jnp.int32, (L, L), 1)
    jstar = jnp.min(jnp.where(sm >= msm, lidx, L), axis=-1, keepdims=True)
    out_ref[0] = jnp.where(lidx == jstar, msm, 0.0)


def _finish_tc(qs, ks, topk_row):
    b_h, L, d = qs.shape
    return pl.pallas_call(
        functools.partial(_finish_kernel, L=L),
        grid=(b_h,),
        in_specs=[
            pl.BlockSpec((1, L), lambda h: (0, 0)),
            pl.BlockSpec((1, L, d), lambda h: (h, 0, 0)),
            pl.BlockSpec((1, L, d), lambda h: (h, 0, 0)),
        ],
        out_specs=pl.BlockSpec((1, L, L), lambda h: (h, 0, 0)),
        out_shape=jax.ShapeDtypeStruct((b_h, L, L), jnp.float32),
    )(topk_row, qs, ks)


def kernel(q, k, bucket_size, topk):
    b_h, t, d = q.shape
    num_samples, L = bucket_size.shape

    sizes = bucket_size[0].astype(jnp.int32)
    bounds = jnp.cumsum(sizes)
    seg = jnp.searchsorted(bounds, jnp.arange(t, dtype=jnp.int32),
                           side="right").astype(jnp.int32)
    seg2d = seg.reshape(t // _CHUNK, _CHUNK)
    seg_off = (seg2d[None, :, :]
               + (jnp.arange(_NS, dtype=jnp.int32) * L)[:, None, None])
    zeros = jnp.zeros((L, d), jnp.float32)

    qs, ks = _segsum_sc(q, k, seg_off, zeros)
    topk_row = jnp.full((1, L), topk, dtype=jnp.float32)
    return _finish_tc(qs, ks, topk_row)


# speedup vs baseline: 10.4625x; 3.6242x over previous
import functools

import jax
import jax.numpy as jnp
from jax import lax
from jax.experimental import pallas as pl
from jax.experimental.pallas import tpu as pltpu
from jax.experimental.pallas import tpu_sc as plsc

_DIM = 128
_NC = 2
_NS = 16
_CHUNK = 64
_NBUF = 4


def _segsum_body(q_hbm, k_hbm, seg_hbm, z_hbm, qs_hbm, ks_hbm,
                 idx_v, b0, b1, b2, b3, accq_sh, acck_sh, s0, s1, s2, s3):
    c = lax.axis_index("c")
    s = lax.axis_index("s")
    wid = s * _NC + c
    bufs = (b0, b1, b2, b3)
    sems = (s0, s1, s2, s3)

    pltpu.sync_copy(seg_hbm.at[s], idx_v)
    pltpu.sync_copy(z_hbm, accq_sh.at[pl.ds(s * 128, 128)])
    pltpu.sync_copy(z_hbm, acck_sh.at[pl.ds(s * 128, 128)])

    n_chunks = q_hbm.shape[1] // _CHUNK
    full_groups = n_chunks // _NBUF
    tail = n_chunks % _NBUF

    def stream_tensor(x_hbm, acc_sh):
        for b in range(_NBUF):
            pltpu.async_copy(x_hbm.at[wid, pl.ds(b * _CHUNK, _CHUNK)],
                             bufs[b], sems[b])

        def outer(g, carry):
            for b in range(_NBUF):
                cc = g * _NBUF + b
                pltpu.make_async_copy(
                    x_hbm.at[wid, pl.ds(cc * _CHUNK, _CHUNK)],
                    bufs[b], sems[b]).wait()
                pltpu.sync_copy(bufs[b], acc_sh.at[idx_v.at[cc]], add=True)
                nxt = cc + _NBUF

                @pl.when(nxt < n_chunks)
                def _():
                    pltpu.async_copy(x_hbm.at[wid, pl.ds(nxt * _CHUNK, _CHUNK)],
                                     bufs[b], sems[b])
            return carry

        lax.fori_loop(0, full_groups, outer, 0)
        for b in range(tail):
            cc = full_groups * _NBUF + b
            pltpu.make_async_copy(
                x_hbm.at[wid, pl.ds(cc * _CHUNK, _CHUNK)],
                bufs[b], sems[b]).wait()
            pltpu.sync_copy(bufs[b], acc_sh.at[idx_v.at[cc]], add=True)

    stream_tensor(q_hbm, accq_sh)
    stream_tensor(k_hbm, acck_sh)

    pltpu.sync_copy(accq_sh.at[pl.ds(s * 128, 128)], qs_hbm.at[wid])
    pltpu.sync_copy(acck_sh.at[pl.ds(s * 128, 128)], ks_hbm.at[wid])


def _segsum_sc(q, k, seg_off, zeros):
    b_h, t, d = q.shape
    L = 128
    mesh = plsc.VectorSubcoreMesh(core_axis_name="c", subcore_axis_name="s")
    f = pl.kernel(
        _segsum_body,
        out_type=[jax.ShapeDtypeStruct((b_h, L, d), jnp.float32),
                  jax.ShapeDtypeStruct((b_h, L, d), jnp.float32)],
        mesh=mesh,
        scratch_types=[
            pltpu.VMEM((t // _CHUNK, _CHUNK), jnp.int32),
            pltpu.VMEM((_CHUNK, d), jnp.float32),
            pltpu.VMEM((_CHUNK, d), jnp.float32),
            pltpu.VMEM((_CHUNK, d), jnp.float32),
            pltpu.VMEM((_CHUNK, d), jnp.float32),
            pltpu.VMEM_SHARED((_NS * L, d), jnp.float32),
            pltpu.VMEM_SHARED((_NS * L, d), jnp.float32),
            pltpu.SemaphoreType.DMA,
            pltpu.SemaphoreType.DMA,
            pltpu.SemaphoreType.DMA,
            pltpu.SemaphoreType.DMA,
        ],
    )
    return f(q, k, seg_off, zeros)


def _meta_kernel(sizes_ref, segoff_ref, *, L, n_chunks):
    pos = (jax.lax.broadcasted_iota(jnp.int32, (n_chunks, _CHUNK), 0) * _CHUNK
           + jax.lax.broadcasted_iota(jnp.int32, (n_chunks, _CHUNK), 1))
    seg = jnp.zeros((n_chunks, _CHUNK), jnp.int32)
    tot = sizes_ref[0, 0] * 0
    for l in range(L):
        tot = tot + sizes_ref[0, l]
        seg = seg + (pos >= tot).astype(jnp.int32)
    for s in range(_NS):
        segoff_ref[s] = seg + s * L


def _meta_tc(sizes, t):
    num_samples, L = sizes.shape
    n_chunks = t // _CHUNK
    return pl.pallas_call(
        functools.partial(_meta_kernel, L=L, n_chunks=n_chunks),
        in_specs=[pl.BlockSpec((1, L), lambda: (0, 0))],
        out_specs=pl.BlockSpec((_NS, n_chunks, _CHUNK), lambda: (0, 0, 0)),
        out_shape=jax.ShapeDtypeStruct((_NS, n_chunks, _CHUNK), jnp.int32),
    )(sizes)


def _finish_kernel(topk_ref, qs_ref, ks_ref, out_ref, *, L):
    qs = qs_ref[0].astype(jnp.bfloat16)
    ks = ks_ref[0].astype(jnp.bfloat16)
    R = jax.lax.dot_general(qs, ks, (((1,), (1,)), ((), ())),
                            preferred_element_type=jnp.float32)
    R = R * jnp.float32(_DIM ** -0.5)
    R = R * topk_ref[...]
    m = jnp.max(R, axis=-1, keepdims=True)
    e = jnp.exp(R - m)
    ssum = jnp.sum(e, axis=-1, keepdims=True)
    sm = e / ssum
    msm = jnp.max(sm, axis=-1, keepdims=True)
    lidx = jax.lax.broadcasted_iota(jnp.int32, (L, L), 1)
    jstar = jnp.min(jnp.where(sm >= msm, lidx, L), axis=-1, keepdims=True)
    out_ref[0] = jnp.where(lidx == jstar, msm, 0.0)


def _finish_tc(qs, ks, topk_row):
    b_h, L, d = qs.shape
    return pl.pallas_call(
        functools.partial(_finish_kernel, L=L),
        grid=(b_h,),
        in_specs=[
            pl.BlockSpec((1, L), lambda h: (0, 0)),
            pl.BlockSpec((1, L, d), lambda h: (h, 0, 0)),
            pl.BlockSpec((1, L, d), lambda h: (h, 0, 0)),
        ],
        out_specs=pl.BlockSpec((1, L, L), lambda h: (h, 0, 0)),
        out_shape=jax.ShapeDtypeStruct((b_h, L, L), jnp.float32),
    )(topk_row, qs, ks)


def kernel(q, k, bucket_size, topk):
    b_h, t, d = q.shape
    num_samples, L = bucket_size.shape

    seg_off = _meta_tc(bucket_size.astype(jnp.int32), t)
    zeros = jnp.zeros((L, d), jnp.float32)

    qs, ks = _segsum_sc(q, k, seg_off, zeros)
    topk_row = jnp.full((1, L), topk, dtype=jnp.float32)
    return _finish_tc(qs, ks, topk_row)
